# Initial kernel scaffold; baseline (speedup 1.0000x reference)
import jax
import jax.numpy as jnp
from jax.experimental import pallas as pl


def kernel(positions, node_attrs, edge_index, shifts, W_embed, Ra0, Rb0, Ws0, Wk0, Ra1, Rb1, Ws1, Wk1, W_node, W_e1, W_erad, W_e2):
    raise NotImplementedError("write your pallas kernel here")


# speedup vs baseline: 2.3815x; 2.3815x over previous
import functools

import jax
import jax.numpy as jnp
import numpy as np
from jax import lax
from jax.experimental import pallas as pl
from jax.experimental.pallas import tpu as pltpu
from jax.experimental.pallas import tpu_sc as plsc

_NC = 2
_NS = 16
_NW = _NC * _NS
_RCUT = 5.0


def _mesh():
  return plsc.VectorSubcoreMesh(
      core_axis_name="c", subcore_axis_name="s",
      num_cores=_NC, num_subcores=_NS)


def _silu(x):
  return x * (1.0 / (1.0 + jnp.exp(-x)))


def _sc_gather(table, idx, chunk):
  e, = idx.shape
  _, d = table.shape
  bpw = e // _NW
  nchunks = bpw // chunk
  assert bpw * _NW == e and nchunks * chunk == bpw and chunk % 8 == 0

  def body(table_hbm, idx_hbm, out_hbm, idx_v, rows_v, sem):
    wid = lax.axis_index("s") * _NC + lax.axis_index("c")
    base = wid * bpw

    def step(i, carry):
      off = base + i * chunk
      pltpu.sync_copy(idx_hbm.at[pl.ds(off, chunk)], idx_v)
      pltpu.async_copy(table_hbm.at[idx_v], rows_v, sem).wait()
      pltpu.sync_copy(rows_v, out_hbm.at[pl.ds(off, chunk)])
      return carry

    lax.fori_loop(0, nchunks, step, 0)

  return pl.kernel(
      body,
      out_type=jax.ShapeDtypeStruct((e, d), jnp.float32),
      mesh=_mesh(),
      scratch_types=[
          pltpu.VMEM((chunk,), jnp.int32),
          pltpu.VMEM((chunk, d), jnp.float32),
          pltpu.SemaphoreType.DMA,
      ],
  )(table, idx)


def _sc_scatter_add(msg, dst, n, chunk):
  e, d = msg.shape
  bpw = e // _NW
  nchunks = bpw // chunk
  np_ = ((n + 8 * _NS - 1) // (8 * _NS)) * (8 * _NS)
  rows_t = np_ // _NS
  assert nchunks * chunk == bpw

  def body(msg_hbm, dst_hbm, zero_hbm, out_hbm, idx_v, rows_v, acc_sh, sem):
    c = lax.axis_index("c")
    s = lax.axis_index("s")
    r0 = s * rows_t
    pltpu.sync_copy(zero_hbm.at[pl.ds(r0, rows_t)], acc_sh.at[pl.ds(r0, rows_t)])
    plsc.subcore_barrier()
    wid = s * _NC + c
    base = wid * bpw

    def step(i, carry):
      off = base + i * chunk
      pltpu.sync_copy(dst_hbm.at[pl.ds(off, chunk)], idx_v)
      pltpu.sync_copy(msg_hbm.at[pl.ds(off, chunk)], rows_v)
      pltpu.sync_copy(rows_v, acc_sh.at[idx_v], add=True)
      return carry

    lax.fori_loop(0, nchunks, step, 0)
    plsc.subcore_barrier()
    pltpu.sync_copy(acc_sh.at[pl.ds(r0, rows_t)],
                    out_hbm.at[c, pl.ds(r0, rows_t)])

  zero = jnp.zeros((np_, d), jnp.float32)
  parts = pl.kernel(
      body,
      out_type=jax.ShapeDtypeStruct((2, np_, d), jnp.float32),
      mesh=_mesh(),
      scratch_types=[
          pltpu.VMEM((chunk,), jnp.int32),
          pltpu.VMEM((chunk, d), jnp.float32),
          pltpu.VMEM_SHARED((np_, d), jnp.float32),
          pltpu.SemaphoreType.DMA,
      ],
  )(msg, dst, zero)
  return parts[:, :n, :]


def _edge_feats(psg, pdg, shf, ra0, rb0, ra1, rb1, be=6400):
  e = psg.shape[1]

  def body(ps_ref, pd_ref, sh_ref, ra0_ref, rb0_ref, ra1_ref, rb1_ref,
           ef_ref, g_ref):
    vx = pd_ref[0:1, :] - ps_ref[0:1, :] + sh_ref[0:1, :]
    vy = pd_ref[1:2, :] - ps_ref[1:2, :] + sh_ref[1:2, :]
    vz = pd_ref[2:3, :] - ps_ref[2:3, :] + sh_ref[2:3, :]
    r2 = vx * vx + vy * vy + vz * vz
    r = jnp.sqrt(r2 + 1e-12)
    rinv = 1.0 / r
    ux = vx * rinv
    uy = vy * rinv
    uz = vz * rinv
    sh = jnp.concatenate([
        jnp.full_like(ux, 0.28209479),
        0.48860251 * uy,
        0.48860251 * uz,
        0.48860251 * ux,
        1.09254843 * ux * uy,
        1.09254843 * uy * uz,
        0.31539157 * (3.0 * uz * uz - 1.0),
        1.09254843 * ux * uz,
        0.54627421 * (ux * ux - uy * uy),
    ], axis=0)
    scale = float(np.sqrt(2.0 / _RCUT))
    wr = float(np.pi / _RCUT)
    ef = jnp.concatenate(
        [scale * jnp.sin((float(k) * wr) * r) * rinv for k in range(1, 9)],
        axis=0)
    rc = jnp.minimum(r * (1.0 / _RCUT), 1.0)
    fc = 0.5 * (jnp.cos(float(np.pi) * rc) + 1.0)
    fc = fc * (r < _RCUT).astype(jnp.float32)
    ef = ef * fc
    ef_ref[...] = ef
    g_rows = []
    for ra_ref, rb_ref in ((ra0_ref, rb0_ref), (ra1_ref, rb1_ref)):
      t = _silu(lax.dot(ra_ref[...], ef, preferred_element_type=jnp.float32))
      ew = lax.dot(rb_ref[...], t, preferred_element_type=jnp.float32)
      g_rows.append(jnp.sum(sh * ew, axis=0, keepdims=True))
    g_rows.append(jnp.zeros((6, ef.shape[1]), jnp.float32))
    g_ref[...] = jnp.concatenate(g_rows, axis=0)

  return pl.pallas_call(
      body,
      grid=(e // be,),
      in_specs=[
          pl.BlockSpec((16, be), lambda i: (0, i)),
          pl.BlockSpec((16, be), lambda i: (0, i)),
          pl.BlockSpec((8, be), lambda i: (0, i)),
          pl.BlockSpec((32, 8), lambda i: (0, 0)),
          pl.BlockSpec((9, 32), lambda i: (0, 0)),
          pl.BlockSpec((32, 8), lambda i: (0, 0)),
          pl.BlockSpec((9, 32), lambda i: (0, 0)),
      ],
      out_specs=[
          pl.BlockSpec((8, be), lambda i: (0, i)),
          pl.BlockSpec((8, be), lambda i: (0, i)),
      ],
      out_shape=[
          jax.ShapeDtypeStruct((8, e), jnp.float32),
          jax.ShapeDtypeStruct((8, e), jnp.float32),
      ],
  )(psg, pdg, shf, ra0, rb0, ra1, rb1)


def _tc_matmul(x, w, bn=2000):
  n, k = x.shape
  m = w.shape[1]

  def body(x_ref, w_ref, o_ref):
    o_ref[...] = lax.dot(x_ref[...], w_ref[...],
                         preferred_element_type=jnp.float32)

  return pl.pallas_call(
      body,
      grid=(n // bn,),
      in_specs=[
          pl.BlockSpec((bn, k), lambda i: (i, 0)),
          pl.BlockSpec((k, m), lambda i: (0, 0)),
      ],
      out_specs=pl.BlockSpec((bn, m), lambda i: (i, 0)),
      out_shape=jax.ShapeDtypeStruct((n, m), jnp.float32),
  )(x, w)


def _tc_update(parts, h, ws, wk, bn=2000):
  n, d = h.shape

  def body(p_ref, h_ref, ws_ref, wk_ref, o_ref):
    agg = p_ref[0] + p_ref[1]
    o_ref[...] = _silu(
        lax.dot(agg, ws_ref[...], preferred_element_type=jnp.float32)
        + lax.dot(h_ref[...], wk_ref[...], preferred_element_type=jnp.float32))

  return pl.pallas_call(
      body,
      grid=(n // bn,),
      in_specs=[
          pl.BlockSpec((2, bn, d), lambda i: (0, i, 0)),
          pl.BlockSpec((bn, d), lambda i: (i, 0)),
          pl.BlockSpec((d, d), lambda i: (0, 0)),
          pl.BlockSpec((d, d), lambda i: (0, 0)),
      ],
      out_specs=pl.BlockSpec((bn, d), lambda i: (i, 0)),
      out_shape=jax.ShapeDtypeStruct((n, d), jnp.float32),
  )(parts, h, ws, wk)


def _tc_scale(hs, gcol, be=6400):
  e, d = hs.shape

  def body(h_ref, g_ref, o_ref):
    o_ref[...] = h_ref[...] * g_ref[...]

  return pl.pallas_call(
      body,
      grid=(e // be,),
      in_specs=[
          pl.BlockSpec((be, d), lambda i: (i, 0)),
          pl.BlockSpec((be, 1), lambda i: (i, 0)),
      ],
      out_specs=pl.BlockSpec((be, d), lambda i: (i, 0)),
      out_shape=jax.ShapeDtypeStruct((e, d), jnp.float32),
  )(hs, gcol)


def _tc_readout_node(h1, h2, wn1, wn2, wa1, wa2, wb1, wb2, bn=2000):
  n, d = h1.shape
  ld = wn1.shape[1]
  eh = wa1.shape[1]

  def body(h1_ref, h2_ref, wn1_ref, wn2_ref, wa1_ref, wa2_ref, wb1_ref,
           wb2_ref, nl_ref, p0_ref, p1_ref):
    h1v = h1_ref[...]
    h2v = h2_ref[...]

    def mm(a, b):
      return lax.dot(a, b, preferred_element_type=jnp.float32)

    nl_ref[...] = mm(h1v, wn1_ref[...]) + mm(h2v, wn2_ref[...])
    p0_ref[...] = mm(h1v, wa1_ref[...]) + mm(h2v, wa2_ref[...])
    p1_ref[...] = mm(h1v, wb1_ref[...]) + mm(h2v, wb2_ref[...])

  return pl.pallas_call(
      body,
      grid=(n // bn,),
      in_specs=[
          pl.BlockSpec((bn, d), lambda i: (i, 0)),
          pl.BlockSpec((bn, d), lambda i: (i, 0)),
          pl.BlockSpec((d, ld), lambda i: (0, 0)),
          pl.BlockSpec((d, ld), lambda i: (0, 0)),
          pl.BlockSpec((d, eh), lambda i: (0, 0)),
          pl.BlockSpec((d, eh), lambda i: (0, 0)),
          pl.BlockSpec((d, eh), lambda i: (0, 0)),
          pl.BlockSpec((d, eh), lambda i: (0, 0)),
      ],
      out_specs=[
          pl.BlockSpec((bn, ld), lambda i: (i, 0)),
          pl.BlockSpec((bn, eh), lambda i: (i, 0)),
          pl.BlockSpec((bn, eh), lambda i: (i, 0)),
      ],
      out_shape=[
          jax.ShapeDtypeStruct((n, ld), jnp.float32),
          jax.ShapeDtypeStruct((n, eh), jnp.float32),
          jax.ShapeDtypeStruct((n, eh), jnp.float32),
      ],
  )(h1, h2, wn1, wn2, wa1, wa2, wb1, wb2)


def _tc_readout_edge(p0g, p1g, ef_rows, w_erad, w_e2, be=6400):
  e, eh = p0g.shape
  nr = w_erad.shape[0]
  ld = w_e2.shape[1]

  def body(p0_ref, p1_ref, ef_ref, wr_ref, w2_ref, o_ref):
    x = p0_ref[...] + p1_ref[...] + lax.dot(
        ef_ref[...], wr_ref[...], preferred_element_type=jnp.float32)
    o_ref[...] = lax.dot(_silu(x), w2_ref[...],
                         preferred_element_type=jnp.float32)

  return pl.pallas_call(
      body,
      grid=(e // be,),
      in_specs=[
          pl.BlockSpec((be, eh), lambda i: (i, 0)),
          pl.BlockSpec((be, eh), lambda i: (i, 0)),
          pl.BlockSpec((be, nr), lambda i: (i, 0)),
          pl.BlockSpec((nr, eh), lambda i: (0, 0)),
          pl.BlockSpec((eh, ld), lambda i: (0, 0)),
      ],
      out_specs=pl.BlockSpec((be, ld), lambda i: (i, 0)),
      out_shape=jax.ShapeDtypeStruct((e, ld), jnp.float32),
  )(p0g, p1g, ef_rows, w_erad, w_e2)


def kernel(positions, node_attrs, edge_index, shifts, W_embed, Ra0, Rb0, Ws0,
           Wk0, Ra1, Rb1, Ws1, Wk1, W_node, W_e1, W_erad, W_e2):
  n = positions.shape[0]
  d = W_embed.shape[1]
  src = edge_index[0]
  dst = edge_index[1]

  pos_pad = jnp.pad(positions, ((0, 0), (0, 125)))
  psg = _sc_gather(pos_pad, src, 200)[:, :16]
  pdg = _sc_gather(pos_pad, dst, 200)[:, :16]
  shf = jnp.pad(shifts, ((0, 0), (0, 5))).T

  ef_t, g_t = _edge_feats(psg.T, pdg.T, shf, Ra0.T, Rb0.T, Ra1.T, Rb1.T)
  g0 = g_t[0:1, :].T
  g1 = g_t[1:2, :].T
  ef_rows = ef_t.T

  h0 = _tc_matmul(node_attrs, W_embed)

  hs0 = _sc_gather(h0, src, 200)
  msg0 = _tc_scale(hs0, g0)
  parts0 = _sc_scatter_add(msg0, dst, n, 200)
  h1 = _tc_update(parts0, h0, Ws0, Wk0)

  hs1 = _sc_gather(h1, src, 200)
  msg1 = _tc_scale(hs1, g1)
  parts1 = _sc_scatter_add(msg1, dst, n, 200)
  h2 = _tc_update(parts1, h1, Ws1, Wk1)

  nl, p0, p1 = _tc_readout_node(
      h1, h2,
      W_node[:d], W_node[d:],
      W_e1[:d], W_e1[d:2 * d], W_e1[2 * d:3 * d], W_e1[3 * d:])

  p0g = _sc_gather(p0, src, 200)
  p1g = _sc_gather(p1, dst, 200)
  el = _tc_readout_edge(p0g, p1g, ef_rows, W_erad, W_e2)
  return jnp.concatenate([nl, el], axis=0)


# speedup vs baseline: 3.4346x; 1.4422x over previous
import functools

import jax
import jax.numpy as jnp
import numpy as np
from jax import lax
from jax.experimental import pallas as pl
from jax.experimental.pallas import tpu as pltpu
from jax.experimental.pallas import tpu_sc as plsc

_NC = 2
_NS = 16
_NW = _NC * _NS
_RCUT = 5.0


def _mesh():
  return plsc.VectorSubcoreMesh(
      core_axis_name="c", subcore_axis_name="s",
      num_cores=_NC, num_subcores=_NS)


def _silu(x):
  return x * (1.0 / (1.0 + jnp.exp(-x)))


def _sc_gather(table, idx, chunk):
  e, = idx.shape
  _, d = table.shape
  bpw = e // _NW
  nchunks = bpw // chunk
  assert bpw * _NW == e and nchunks * chunk == bpw and chunk % 8 == 0

  def body(table_hbm, idx_hbm, out_hbm, idx_v, rows_v, sem):
    wid = lax.axis_index("s") * _NC + lax.axis_index("c")
    base = wid * bpw

    def step(i, carry):
      off = base + i * chunk
      pltpu.sync_copy(idx_hbm.at[pl.ds(off, chunk)], idx_v)
      pltpu.async_copy(table_hbm.at[idx_v], rows_v, sem).wait()
      pltpu.sync_copy(rows_v, out_hbm.at[pl.ds(off, chunk)])
      return carry

    lax.fori_loop(0, nchunks, step, 0)

  return pl.kernel(
      body,
      out_type=jax.ShapeDtypeStruct((e, d), jnp.float32),
      mesh=_mesh(),
      scratch_types=[
          pltpu.VMEM((chunk,), jnp.int32),
          pltpu.VMEM((chunk, d), jnp.float32),
          pltpu.SemaphoreType.DMA,
      ],
  )(table, idx)


def _sc_gather_pos(pos_x, pos_y, pos_z, src, dst):
  e, = src.shape
  bpw = e // _NW

  def body(px_hbm, py_hbm, pz_hbm, src_hbm, dst_hbm,
           sx_hbm, sy_hbm, sz_hbm, dx_hbm, dy_hbm, dz_hbm,
           sidx_v, didx_v, b0, b1, b2, b3, b4, b5, sem):
    c = lax.axis_index("c")
    s = lax.axis_index("s")
    wid = s * _NC + c
    off = wid * bpw
    pltpu.sync_copy(src_hbm.at[pl.ds(off, bpw)], sidx_v)
    pltpu.sync_copy(dst_hbm.at[pl.ds(off, bpw)], didx_v)
    descs = [
        pltpu.async_copy(px_hbm.at[sidx_v], b0, sem),
        pltpu.async_copy(py_hbm.at[sidx_v], b1, sem),
        pltpu.async_copy(pz_hbm.at[sidx_v], b2, sem),
        pltpu.async_copy(px_hbm.at[didx_v], b3, sem),
        pltpu.async_copy(py_hbm.at[didx_v], b4, sem),
        pltpu.async_copy(pz_hbm.at[didx_v], b5, sem),
    ]
    for dsc in descs:
      dsc.wait()
    for buf, out in ((b0, sx_hbm), (b1, sy_hbm), (b2, sz_hbm),
                     (b3, dx_hbm), (b4, dy_hbm), (b5, dz_hbm)):
      pltpu.sync_copy(buf, out.at[pl.ds(off, bpw)])

  ot = jax.ShapeDtypeStruct((e,), jnp.float32)
  return pl.kernel(
      body,
      out_type=[ot] * 6,
      mesh=_mesh(),
      scratch_types=[
          pltpu.VMEM((bpw,), jnp.int32),
          pltpu.VMEM((bpw,), jnp.int32),
      ] + [pltpu.VMEM((bpw,), jnp.float32)] * 6 + [pltpu.SemaphoreType.DMA],
  )(pos_x, pos_y, pos_z, src, dst)


def _sc_layer(h, g, src, dst, n, chunk=200):
  e, = g.shape
  d = h.shape[1]
  bpw = e // _NW
  nchunks = bpw // chunk
  np_ = ((n + 8 * _NS - 1) // (8 * _NS)) * (8 * _NS)
  rows_t = np_ // _NS
  nseg = d // 16
  assert nchunks * chunk == bpw

  def body(h_hbm, g_hbm, src_hbm, dst_hbm, zero_hbm, out_hbm,
           sidx_v, didx_v, g_v, rows_v, acc_sh, sem):
    c = lax.axis_index("c")
    s = lax.axis_index("s")
    r0 = s * rows_t
    pltpu.sync_copy(zero_hbm.at[pl.ds(r0, rows_t)], acc_sh.at[pl.ds(r0, rows_t)])
    plsc.subcore_barrier()
    wid = s * _NC + c
    base = wid * bpw

    def step(i, carry):
      off = base + i * chunk
      pltpu.sync_copy(src_hbm.at[pl.ds(off, chunk)], sidx_v)
      pltpu.sync_copy(dst_hbm.at[pl.ds(off, chunk)], didx_v)
      pltpu.sync_copy(g_hbm.at[pl.ds(off, chunk)], g_v.at[pl.ds(0, chunk)])
      pltpu.async_copy(h_hbm.at[sidx_v], rows_v, sem).wait()

      def scale_row(j, carry2):
        gs = jnp.full((16,), g_v[pl.ds(j, 16)][0])
        for k in range(nseg):
          rows_v[j, pl.ds(k * 16, 16)] = rows_v[j, pl.ds(k * 16, 16)] * gs
        return carry2

      lax.fori_loop(0, chunk, scale_row, 0)
      pltpu.sync_copy(rows_v, acc_sh.at[didx_v], add=True)
      return carry

    lax.fori_loop(0, nchunks, step, 0)
    plsc.subcore_barrier()
    pltpu.sync_copy(acc_sh.at[pl.ds(r0, rows_t)],
                    out_hbm.at[c, pl.ds(r0, rows_t)])

  zero = jnp.zeros((np_, d), jnp.float32)
  parts = pl.kernel(
      body,
      out_type=jax.ShapeDtypeStruct((2, np_, d), jnp.float32),
      mesh=_mesh(),
      scratch_types=[
          pltpu.VMEM((chunk,), jnp.int32),
          pltpu.VMEM((chunk,), jnp.int32),
          pltpu.VMEM((chunk + 16,), jnp.float32),
          pltpu.VMEM((chunk, d), jnp.float32),
          pltpu.VMEM_SHARED((np_, d), jnp.float32),
          pltpu.SemaphoreType.DMA,
      ],
  )(h, g, src, dst, zero)
  return parts[:, :n, :]


def _sc_scatter_add(msg, dst, n, chunk):
  e, d = msg.shape
  bpw = e // _NW
  nchunks = bpw // chunk
  np_ = ((n + 8 * _NS - 1) // (8 * _NS)) * (8 * _NS)
  rows_t = np_ // _NS
  assert nchunks * chunk == bpw

  def body(msg_hbm, dst_hbm, zero_hbm, out_hbm, idx_v, rows_v, acc_sh, sem):
    c = lax.axis_index("c")
    s = lax.axis_index("s")
    r0 = s * rows_t
    pltpu.sync_copy(zero_hbm.at[pl.ds(r0, rows_t)], acc_sh.at[pl.ds(r0, rows_t)])
    plsc.subcore_barrier()
    wid = s * _NC + c
    base = wid * bpw

    def step(i, carry):
      off = base + i * chunk
      pltpu.sync_copy(dst_hbm.at[pl.ds(off, chunk)], idx_v)
      pltpu.sync_copy(msg_hbm.at[pl.ds(off, chunk)], rows_v)
      pltpu.sync_copy(rows_v, acc_sh.at[idx_v], add=True)
      return carry

    lax.fori_loop(0, nchunks, step, 0)
    plsc.subcore_barrier()
    pltpu.sync_copy(acc_sh.at[pl.ds(r0, rows_t)],
                    out_hbm.at[c, pl.ds(r0, rows_t)])

  zero = jnp.zeros((np_, d), jnp.float32)
  parts = pl.kernel(
      body,
      out_type=jax.ShapeDtypeStruct((2, np_, d), jnp.float32),
      mesh=_mesh(),
      scratch_types=[
          pltpu.VMEM((chunk,), jnp.int32),
          pltpu.VMEM((chunk, d), jnp.float32),
          pltpu.VMEM_SHARED((np_, d), jnp.float32),
          pltpu.SemaphoreType.DMA,
      ],
  )(msg, dst, zero)
  return parts[:, :n, :]


def _edge_feats(pcomp, shf, ra0, rb0, ra1, rb1, be=6400):
  e = pcomp.shape[1]

  def body(p_ref, sh_ref, ra0_ref, rb0_ref, ra1_ref, rb1_ref,
           ef_ref, g_ref):
    vx = p_ref[3:4, :] - p_ref[0:1, :] + sh_ref[0:1, :]
    vy = p_ref[4:5, :] - p_ref[1:2, :] + sh_ref[1:2, :]
    vz = p_ref[5:6, :] - p_ref[2:3, :] + sh_ref[2:3, :]
    r2 = vx * vx + vy * vy + vz * vz
    r = jnp.sqrt(r2 + 1e-12)
    rinv = 1.0 / r
    ux = vx * rinv
    uy = vy * rinv
    uz = vz * rinv
    sh = jnp.concatenate([
        jnp.full_like(ux, 0.28209479),
        0.48860251 * uy,
        0.48860251 * uz,
        0.48860251 * ux,
        1.09254843 * ux * uy,
        1.09254843 * uy * uz,
        0.31539157 * (3.0 * uz * uz - 1.0),
        1.09254843 * ux * uz,
        0.54627421 * (ux * ux - uy * uy),
    ], axis=0)
    scale = float(np.sqrt(2.0 / _RCUT))
    wr = float(np.pi / _RCUT)
    ef = jnp.concatenate(
        [scale * jnp.sin((float(k) * wr) * r) * rinv for k in range(1, 9)],
        axis=0)
    rc = jnp.minimum(r * (1.0 / _RCUT), 1.0)
    fc = 0.5 * (jnp.cos(float(np.pi) * rc) + 1.0)
    fc = fc * (r < _RCUT).astype(jnp.float32)
    ef = ef * fc
    ef_ref[...] = ef
    g_rows = []
    for ra_ref, rb_ref in ((ra0_ref, rb0_ref), (ra1_ref, rb1_ref)):
      t = _silu(lax.dot(ra_ref[...], ef, preferred_element_type=jnp.float32))
      ew = lax.dot(rb_ref[...], t, preferred_element_type=jnp.float32)
      g_rows.append(jnp.sum(sh * ew, axis=0, keepdims=True))
    g_rows.append(jnp.zeros((6, ef.shape[1]), jnp.float32))
    g_ref[...] = jnp.concatenate(g_rows, axis=0)

  return pl.pallas_call(
      body,
      grid=(e // be,),
      in_specs=[
          pl.BlockSpec((6, be), lambda i: (0, i)),
          pl.BlockSpec((8, be), lambda i: (0, i)),
          pl.BlockSpec((32, 8), lambda i: (0, 0)),
          pl.BlockSpec((9, 32), lambda i: (0, 0)),
          pl.BlockSpec((32, 8), lambda i: (0, 0)),
          pl.BlockSpec((9, 32), lambda i: (0, 0)),
      ],
      out_specs=[
          pl.BlockSpec((8, be), lambda i: (0, i)),
          pl.BlockSpec((8, be), lambda i: (0, i)),
      ],
      out_shape=[
          jax.ShapeDtypeStruct((8, e), jnp.float32),
          jax.ShapeDtypeStruct((8, e), jnp.float32),
      ],
  )(pcomp, shf, ra0, rb0, ra1, rb1)


def _tc_matmul(x, w, bn=2000):
  n, k = x.shape
  m = w.shape[1]

  def body(x_ref, w_ref, o_ref):
    o_ref[...] = lax.dot(x_ref[...], w_ref[...],
                         preferred_element_type=jnp.float32)

  return pl.pallas_call(
      body,
      grid=(n // bn,),
      in_specs=[
          pl.BlockSpec((bn, k), lambda i: (i, 0)),
          pl.BlockSpec((k, m), lambda i: (0, 0)),
      ],
      out_specs=pl.BlockSpec((bn, m), lambda i: (i, 0)),
      out_shape=jax.ShapeDtypeStruct((n, m), jnp.float32),
  )(x, w)


def _tc_update(parts, h, ws, wk, bn=2000):
  n, d = h.shape

  def body(p_ref, h_ref, ws_ref, wk_ref, o_ref):
    agg = p_ref[0] + p_ref[1]
    o_ref[...] = _silu(
        lax.dot(agg, ws_ref[...], preferred_element_type=jnp.float32)
        + lax.dot(h_ref[...], wk_ref[...], preferred_element_type=jnp.float32))

  return pl.pallas_call(
      body,
      grid=(n // bn,),
      in_specs=[
          pl.BlockSpec((2, bn, d), lambda i: (0, i, 0)),
          pl.BlockSpec((bn, d), lambda i: (i, 0)),
          pl.BlockSpec((d, d), lambda i: (0, 0)),
          pl.BlockSpec((d, d), lambda i: (0, 0)),
      ],
      out_specs=pl.BlockSpec((bn, d), lambda i: (i, 0)),
      out_shape=jax.ShapeDtypeStruct((n, d), jnp.float32),
  )(parts, h, ws, wk)


def _tc_scale(hs, gcol, be=6400):
  e, d = hs.shape

  def body(h_ref, g_ref, o_ref):
    o_ref[...] = h_ref[...] * g_ref[...]

  return pl.pallas_call(
      body,
      grid=(e // be,),
      in_specs=[
          pl.BlockSpec((be, d), lambda i: (i, 0)),
          pl.BlockSpec((be, 1), lambda i: (i, 0)),
      ],
      out_specs=pl.BlockSpec((be, d), lambda i: (i, 0)),
      out_shape=jax.ShapeDtypeStruct((e, d), jnp.float32),
  )(hs, gcol)


def _pack_bf16(even, odd):
  lo = lax.convert_element_type(
      lax.bitcast_convert_type(even.astype(jnp.bfloat16), jnp.uint16),
      jnp.uint32)
  hi = lax.convert_element_type(
      lax.bitcast_convert_type(odd.astype(jnp.bfloat16), jnp.uint16),
      jnp.uint32)
  return lax.bitcast_convert_type(lo | (hi << 16), jnp.float32)


def _unpack_bf16(x):
  u = lax.bitcast_convert_type(x, jnp.uint32)
  even = lax.bitcast_convert_type(u << 16, jnp.float32)
  odd = lax.bitcast_convert_type(u & jnp.uint32(0xFFFF0000), jnp.float32)
  return even, odd


def _tc_readout_node(h1, h2, wn1, wn2, wa1e, wa1o, wa2e, wa2o, wb1e, wb1o,
                     wb2e, wb2o, bn=2000):
  n, d = h1.shape
  ld = wn1.shape[1]
  hd = wa1e.shape[1]

  def body(h1_ref, h2_ref, wn1_ref, wn2_ref, wa1e_ref, wa1o_ref, wa2e_ref,
           wa2o_ref, wb1e_ref, wb1o_ref, wb2e_ref, wb2o_ref,
           nl_ref, p0_ref, p1_ref):
    h1v = h1_ref[...]
    h2v = h2_ref[...]

    def mm(a, b):
      return lax.dot(a, b, preferred_element_type=jnp.float32)

    nl_ref[...] = mm(h1v, wn1_ref[...]) + mm(h2v, wn2_ref[...])
    p0_ref[...] = _pack_bf16(mm(h1v, wa1e_ref[...]) + mm(h2v, wa2e_ref[...]),
                             mm(h1v, wa1o_ref[...]) + mm(h2v, wa2o_ref[...]))
    p1_ref[...] = _pack_bf16(mm(h1v, wb1e_ref[...]) + mm(h2v, wb2e_ref[...]),
                             mm(h1v, wb1o_ref[...]) + mm(h2v, wb2o_ref[...]))

  wspec = pl.BlockSpec((d, hd), lambda i: (0, 0))
  return pl.pallas_call(
      body,
      grid=(n // bn,),
      in_specs=[
          pl.BlockSpec((bn, d), lambda i: (i, 0)),
          pl.BlockSpec((bn, d), lambda i: (i, 0)),
          pl.BlockSpec((d, ld), lambda i: (0, 0)),
          pl.BlockSpec((d, ld), lambda i: (0, 0)),
          wspec, wspec, wspec, wspec, wspec, wspec, wspec, wspec,
      ],
      out_specs=[
          pl.BlockSpec((bn, ld), lambda i: (i, 0)),
          pl.BlockSpec((bn, hd), lambda i: (i, 0)),
          pl.BlockSpec((bn, hd), lambda i: (i, 0)),
      ],
      out_shape=[
          jax.ShapeDtypeStruct((n, ld), jnp.float32),
          jax.ShapeDtypeStruct((n, hd), jnp.float32),
          jax.ShapeDtypeStruct((n, hd), jnp.float32),
      ],
  )(h1, h2, wn1, wn2, wa1e, wa1o, wa2e, wa2o, wb1e, wb1o, wb2e, wb2o)


def _tc_readout_edge(p0g, p1g, ef_rows, w_erad_e, w_erad_o, w_e2_e, w_e2_o,
                     be=6400):
  e, hd = p0g.shape
  nr = w_erad_e.shape[0]
  ld = w_e2_e.shape[1]

  def body(p0_ref, p1_ref, ef_ref, wre_ref, wro_ref, w2e_ref, w2o_ref,
           o_ref):
    p0e, p0o = _unpack_bf16(p0_ref[...])
    p1e, p1o = _unpack_bf16(p1_ref[...])
    efv = ef_ref[...]

    def mm(a, b):
      return lax.dot(a, b, preferred_element_type=jnp.float32)

    ehe = _silu(p0e + p1e + mm(efv, wre_ref[...]))
    eho = _silu(p0o + p1o + mm(efv, wro_ref[...]))
    o_ref[...] = mm(ehe, w2e_ref[...]) + mm(eho, w2o_ref[...])

  return pl.pallas_call(
      body,
      grid=(e // be,),
      in_specs=[
          pl.BlockSpec((be, hd), lambda i: (i, 0)),
          pl.BlockSpec((be, hd), lambda i: (i, 0)),
          pl.BlockSpec((be, nr), lambda i: (i, 0)),
          pl.BlockSpec((nr, hd), lambda i: (0, 0)),
          pl.BlockSpec((nr, hd), lambda i: (0, 0)),
          pl.BlockSpec((hd, ld), lambda i: (0, 0)),
          pl.BlockSpec((hd, ld), lambda i: (0, 0)),
      ],
      out_specs=pl.BlockSpec((be, ld), lambda i: (i, 0)),
      out_shape=jax.ShapeDtypeStruct((e, ld), jnp.float32),
  )(p0g, p1g, ef_rows, w_erad_e, w_erad_o, w_e2_e, w_e2_o)


def kernel(positions, node_attrs, edge_index, shifts, W_embed, Ra0, Rb0, Ws0,
           Wk0, Ra1, Rb1, Ws1, Wk1, W_node, W_e1, W_erad, W_e2):
  n = positions.shape[0]
  d = W_embed.shape[1]
  src = edge_index[0]
  dst = edge_index[1]

  pcomps = _sc_gather_pos(positions[:, 0], positions[:, 1], positions[:, 2],
                          src, dst)
  pcomp = jnp.stack(pcomps, axis=0)
  shf = jnp.pad(shifts, ((0, 0), (0, 5))).T

  ef_t, g_t = _edge_feats(pcomp, shf, Ra0.T, Rb0.T, Ra1.T, Rb1.T)
  g0 = g_t[0]
  g1 = g_t[1]
  ef_rows = ef_t.T

  h0 = _tc_matmul(node_attrs, W_embed)

  parts0 = _sc_layer(h0, g0, src, dst, n)
  h1 = _tc_update(parts0, h0, Ws0, Wk0)
  parts1 = _sc_layer(h1, g1, src, dst, n)
  h2 = _tc_update(parts1, h1, Ws1, Wk1)

  nl, p0p, p1p = _tc_readout_node(
      h1, h2,
      W_node[:d], W_node[d:],
      W_e1[:d, 0::2], W_e1[:d, 1::2],
      W_e1[d:2 * d, 0::2], W_e1[d:2 * d, 1::2],
      W_e1[2 * d:3 * d, 0::2], W_e1[2 * d:3 * d, 1::2],
      W_e1[3 * d:, 0::2], W_e1[3 * d:, 1::2])

  p0g = _sc_gather(p0p, src, 200)
  p1g = _sc_gather(p1p, dst, 200)
  el = _tc_readout_edge(p0g, p1g, ef_rows,
                        W_erad[:, 0::2], W_erad[:, 1::2],
                        W_e2[0::2], W_e2[1::2])
  return jnp.concatenate([nl, el], axis=0)
